# Initial kernel scaffold; baseline (speedup 1.0000x reference)
import jax
import jax.numpy as jnp
from jax.experimental import pallas as pl


def kernel(arg0_1, arg3_1, convert_element_type, convert_element_type_1):
    raise NotImplementedError("write your pallas kernel here")


# speedup vs baseline: 39.8448x; 39.8448x over previous
import functools

import jax
import jax.numpy as jnp
from jax import lax
from jax.experimental import pallas as pl
from jax.experimental.pallas import tpu as pltpu
from jax.experimental.pallas import tpu_sc as plsc

_N = 10000
_E = 320000
_D = 128
_NC = 2
_NS = 16
_RPT = _D // _NS
_EPC = _E // _NC
_G = 5
_CHUNK = _G * 128
_NCHUNKS = _EPC // _CHUNK
_NPAIRS = _NCHUNKS // 2


def _sc_scatter_body(idx_hbm, src_hbm, out_hbm,
                     acc, idx0, idx1, src0, src1,
                     sem_i0, sem_i1, sem_s0, sem_s1):
  c = lax.axis_index("c")
  s = lax.axis_index("s")
  row0 = s * _RPT
  col_base = c * _EPC

  lane = jax.lax.iota(jnp.int32, 16)
  cols = lane & 7
  eoff = lane >> 3
  m_lo = lane < 8
  m_hi = lane >= 8
  zeros16 = jnp.zeros((16,), jnp.float32)


  def blocks(ci):
    col0 = col_base + ci * _CHUNK
    for g in range(_G):
      yield g, pl.ds(col0 + g * 128, 128)

  def start_chunk(ci, idxb, srcb, sem_i, sem_s):
    for g, csl in blocks(ci):
      pltpu.make_async_copy(
          idx_hbm.at[pl.ds(row0, _RPT), csl], idxb.at[g], sem_i).start()
      pltpu.make_async_copy(
          src_hbm.at[pl.ds(row0, _RPT), csl], srcb.at[g], sem_s).start()

  def wait_chunk(ci, idxb, srcb, sem_i, sem_s):
    for g, csl in blocks(ci):
      pltpu.make_async_copy(
          idx_hbm.at[pl.ds(row0, _RPT), csl], idxb.at[g], sem_i).wait()
      pltpu.make_async_copy(
          src_hbm.at[pl.ds(row0, _RPT), csl], srcb.at[g], sem_s).wait()

  def consume(idxb, srcb):
    for g in range(_G):
      gv = jnp.full((16,), g, jnp.int32)

      @pl.loop(0, 64, unroll=4)
      def _(t):
        ecols = eoff + 2 * t
        a = plsc.load_gather(idxb, [gv, cols, ecols])
        v = plsc.load_gather(srcb, [gv, cols, ecols])
        af = jnp.left_shift(a, 3) + cols
        plsc.addupdate_scatter(acc, [af], v, mask=m_lo)
        plsc.addupdate_scatter(acc, [af], v, mask=m_hi)

  start_chunk(0, idx0, src0, sem_i0, sem_s0)

  @pl.loop(0, _N * _RPT // 16, unroll=8)
  def _(t):
    acc[pl.ds(16 * t, 16)] = zeros16

  @pl.loop(0, _NPAIRS)
  def _(g):
    ci = 2 * g
    wait_chunk(ci, idx0, src0, sem_i0, sem_s0)
    start_chunk(ci + 1, idx1, src1, sem_i1, sem_s1)
    consume(idx0, src0)
    wait_chunk(ci + 1, idx1, src1, sem_i1, sem_s1)

    @pl.when(g < _NPAIRS - 1)
    def _():
      start_chunk(ci + 2, idx0, src0, sem_i0, sem_s0)

    consume(idx1, src1)

  pltpu.sync_copy(acc, out_hbm.at[c, s])


_sc_scatter = pl.kernel(
    _sc_scatter_body,
    out_type=jax.ShapeDtypeStruct((_NC, _NS, _N * _RPT), jnp.float32),
    mesh=plsc.VectorSubcoreMesh(core_axis_name="c", subcore_axis_name="s"),
    compiler_params=pltpu.CompilerParams(needs_layout_passes=False),
    scratch_types=[
        pltpu.VMEM((_N * _RPT,), jnp.float32),
        pltpu.VMEM((_G, _RPT, 128), jnp.int32),
        pltpu.VMEM((_G, _RPT, 128), jnp.int32),
        pltpu.VMEM((_G, _RPT, 128), jnp.float32),
        pltpu.VMEM((_G, _RPT, 128), jnp.float32),
        pltpu.SemaphoreType.DMA,
        pltpu.SemaphoreType.DMA,
        pltpu.SemaphoreType.DMA,
        pltpu.SemaphoreType.DMA,
    ],
)


def _transpose_body(idx_ref, src_ref, idx_t_ref, src_t_ref):
  idx_t_ref[...] = idx_ref[...].T
  src_t_ref[...] = src_ref[...].T


_EDGES_PER_TBLOCK = 3200


@jax.jit
def _transpose(arg0_1, convert_element_type_1):
  grid = (_E // _EDGES_PER_TBLOCK,)
  return pl.pallas_call(
      _transpose_body,
      grid=grid,
      in_specs=[
          pl.BlockSpec((_EDGES_PER_TBLOCK, _D), lambda i: (i, 0)),
          pl.BlockSpec((_EDGES_PER_TBLOCK, _D), lambda i: (i, 0)),
      ],
      out_specs=[
          pl.BlockSpec((_D, _EDGES_PER_TBLOCK), lambda i: (0, i)),
          pl.BlockSpec((_D, _EDGES_PER_TBLOCK), lambda i: (0, i)),
      ],
      out_shape=[
          jax.ShapeDtypeStruct((_D, _E), jnp.int32),
          jax.ShapeDtypeStruct((_D, _E), jnp.float32),
      ],
  )(arg0_1, convert_element_type_1)


def _combine_body(arg3_ref, base_ref, part_ref, out_ref):
  denom = base_ref[...] + part_ref[0] + part_ref[1]
  out_ref[...] = arg3_ref[...] / denom


_ROWS_PER_BLOCK = 1000


@jax.jit
def _combine(arg3_1, base, partials):
  grid = (_N // _ROWS_PER_BLOCK,)
  return pl.pallas_call(
      _combine_body,
      grid=grid,
      in_specs=[
          pl.BlockSpec((_ROWS_PER_BLOCK, _D), lambda i: (i, 0)),
          pl.BlockSpec((_ROWS_PER_BLOCK, _D), lambda i: (i, 0)),
          pl.BlockSpec((_NC, _ROWS_PER_BLOCK, _D), lambda i: (0, i, 0)),
      ],
      out_specs=pl.BlockSpec((_ROWS_PER_BLOCK, _D), lambda i: (i, 0)),
      out_shape=jax.ShapeDtypeStruct((_N, _D), jnp.float32),
  )(arg3_1, base, partials)


@jax.jit
def kernel(arg0_1, arg3_1, convert_element_type, convert_element_type_1):
  idx_t, src_t = _transpose(arg0_1, convert_element_type_1)
  partials = _sc_scatter(idx_t, src_t)
  partials = jnp.transpose(
      partials.reshape(_NC, _NS, _N, _RPT), (0, 2, 1, 3)).reshape(_NC, _N, _D)
  out = _combine(arg3_1, convert_element_type, partials)
  return (out,)
